# baseline (speedup 1.0000x reference)
import functools

import jax
import jax.numpy as jnp
from jax import lax
from jax.experimental import pallas as pl
from jax.experimental.pallas import tpu as pltpu
from jax.experimental.pallas import tpu_sc as plsc

_LANES = 16
_EPS = 1e-12
_UNROLL = 8

_DNUMS = lax.GatherDimensionNumbers(
    offset_dims=(), collapsed_slice_dims=(0,), start_index_map=(0,))


def _permute(v, perm):
    return lax.gather(v, perm.reshape(_LANES, 1), _DNUMS, (1,),
                      mode=lax.GatherScatterMode.PROMISE_IN_BOUNDS)


def _lane_sum(v):
    for k in (8, 4, 2, 1):
        v = v + _permute(v, lax.iota(jnp.int32, _LANES) ^ k)
    return v


def _rsqrt(v):
    i = lax.bitcast_convert_type(v, jnp.int32)
    i = jnp.int32(0x5F3759DF) - lax.shift_right_logical(i, 1)
    y = lax.bitcast_convert_type(i, jnp.float32)
    for _ in range(1):
        y = y * (1.5 - 0.5 * v * y * y)
    return y


def _build(N, D, L, n_workers):
    per_w = N // n_workers
    n_chunks = per_w // L
    n_pairs = n_chunks // 2
    n_sl = D // _LANES
    sub = [(st, min(128, L - st)) for st in range(0, L, 128)]
    mesh = plsc.VectorSubcoreMesh(core_axis_name="c", subcore_axis_name="s")
    info = plsc.get_sparse_core_info()
    NC, NS = info.num_cores, info.num_subcores

    @functools.partial(
        pl.kernel,
        mesh=mesh,
        out_type=jax.ShapeDtypeStruct((N, D), jnp.float32),
        scratch_types=[
            pltpu.VMEM((per_w,), jnp.int32),
            pltpu.VMEM((per_w,), jnp.int32),
            pltpu.VMEM((per_w,), jnp.int32),
            pltpu.VMEM((2, D), jnp.float32),
            pltpu.VMEM((L, D), jnp.float32),
            pltpu.VMEM((L, D), jnp.float32),
            pltpu.VMEM((L, D), jnp.float32),
            pltpu.VMEM((L, D), jnp.float32),
            pltpu.VMEM((D,), jnp.float32),
            pltpu.VMEM((D,), jnp.float32),
            pltpu.VMEM_SHARED((2 * L, D), jnp.float32),
            pltpu.SemaphoreType.DMA,
            pltpu.SemaphoreType.DMA,
            pltpu.SemaphoreType.DMA,
            pltpu.SemaphoreType.DMA,
            pltpu.SemaphoreType.DMA,
            pltpu.SemaphoreType.DMA,
            pltpu.SemaphoreType.DMA,
            pltpu.SemaphoreType.DMA,
            pltpu.SemaphoreType.DMA,
            pltpu.SemaphoreType.DMA,
            pltpu.SemaphoreType.DMA,
            pltpu.SemaphoreType.DMA,
        ],
    )
    def k(ids_hbm, tids_hbm, ww_hbm, wt_hbm, wp_hbm, g_hbm, b_hbm, out_hbm,
          widxv, tidxv, cidxv, wtypev, buf0, buf1, buf2, buf3, gv, bv, ptsh,
          sw0, sw1, sw2, sw3, sp0, sp1, sp2, sp3, so0, so1, so2, so3):
        bufs = (buf0, buf1, buf2, buf3)
        sws = (sw0, sw1, sw2, sw3)
        sps = (sp0, sp1, sp2, sp3)
        sos = (so0, so1, so2, so3)
        bufa = buf0
        sid = lax.axis_index("s")
        wid = sid * NC + lax.axis_index("c")
        base = wid * per_w
        pltpu.sync_copy(g_hbm, gv)
        pltpu.sync_copy(b_hbm, bv)
        pltpu.sync_copy(wt_hbm, wtypev)
        pltpu.sync_copy(ids_hbm.at[pl.ds(base, per_w)], widxv)
        pltpu.sync_copy(tids_hbm.at[pl.ds(base, per_w)], tidxv)

        n_blocks = (2 * L) // 8
        for kb in range((n_blocks + NS - 1) // NS):
            bb = sid + NS * kb

            @pl.when(bb < n_blocks)
            def _():
                t = (bb * 8) // L
                p0 = bb * 8 - t * L
                pltpu.sync_copy(wp_hbm.at[pl.ds(p0, 8)], bufa.at[pl.ds(0, 8)])
                for r in range(8):
                    for s in range(n_sl):
                        sl = pl.ds(s * _LANES, _LANES)
                        bufa[r, sl] = bufa[r, sl] + wtypev[t, sl]
                pltpu.sync_copy(bufa.at[pl.ds(0, 8)],
                                ptsh.at[pl.ds(bb * 8, 8)])

        def cidx_body(g, carry):
            g0 = g * _LANES
            pos = lax.rem(jnp.full((_LANES,), g0, jnp.int32)
                          + lax.iota(jnp.int32, _LANES), jnp.int32(L))
            cidxv[pl.ds(g0, _LANES)] = tidxv[pl.ds(g0, _LANES)] * L + pos
            return carry

        lax.fori_loop(0, per_w // _LANES, cidx_body, 0)
        plsc.subcore_barrier()

        def start_word(c, buf, sem):
            for st, ln in sub:
                pltpu.async_copy(
                    ww_hbm.at[widxv.at[pl.ds(c * L + st, ln)]],
                    buf.at[pl.ds(st, ln)], sem)

        def wait_word(buf, sem):
            for st, ln in sub:
                pltpu.make_async_copy(
                    ww_hbm.at[widxv.at[pl.ds(st, ln)]],
                    buf.at[pl.ds(st, ln)], sem).wait()

        def start_pt(c, buf, sem):
            for st, ln in sub:
                pltpu.async_copy(
                    ptsh.at[cidxv.at[pl.ds(c * L + st, ln)]],
                    buf.at[pl.ds(st, ln)], sem, add=True)

        def wait_pt(buf, sem):
            for st, ln in sub:
                pltpu.make_async_copy(
                    ptsh.at[cidxv.at[pl.ds(st, ln)]],
                    buf.at[pl.ds(st, ln)], sem).wait()

        def start_out(c, buf, sem):
            pltpu.async_copy(buf, out_hbm.at[pl.ds(base + c * L, L)], sem)

        def wait_out(buf, sem):
            pltpu.make_async_copy(buf, out_hbm.at[pl.ds(0, L)], sem).wait()

        def compute(buf):
            inv0 = tuple(
                ref[pl.ds(s * _LANES, _LANES)]
                for ref in (gv, bv) for s in range(n_sl))

            def tok_body(g, inv):
                gs, bs = inv[:n_sl], inv[n_sl:]
                for u in range(_UNROLL):
                    i = g * _UNROLL + u
                    sum_v = None
                    ssq_v = None
                    for s in range(n_sl):
                        sl = pl.ds(s * _LANES, _LANES)
                        x = buf[i, sl]
                        sum_v = x if s == 0 else sum_v + x
                        ssq_v = x * x if s == 0 else ssq_v + x * x
                    mean_v = _lane_sum(sum_v) * (1.0 / D)
                    var_v = _lane_sum(ssq_v) * (1.0 / D) - mean_v * mean_v
                    scale_v = _rsqrt(var_v + _EPS)
                    for s in range(n_sl):
                        sl = pl.ds(s * _LANES, _LANES)
                        buf[i, sl] = (buf[i, sl] - mean_v) * scale_v * gs[s] + bs[s]
                return inv

            lax.fori_loop(0, L // _UNROLL, tok_body, inv0)

        last = jnp.int32(n_chunks - 1)
        assert n_chunks % 4 == 0

        start_word(jnp.int32(0), bufs[0], sws[0])
        start_word(jnp.int32(1), bufs[1], sws[1])
        start_word(jnp.int32(2), bufs[2], sws[2])
        wait_word(bufs[0], sws[0])
        start_pt(jnp.int32(0), bufs[0], sps[0])

        def slot_body(t, carry):
            for u in range(4):
                c = t * 4 + u
                un, ug = (u + 1) % 4, (u + 3) % 4
                wait_word(bufs[un], sws[un])
                start_pt(jnp.minimum(c + 1, last), bufs[un], sps[un])
                wait_pt(bufs[u], sps[u])
                compute(bufs[u])
                start_out(c, bufs[u], sos[u])

                @pl.when(c >= 1)
                def _():
                    wait_out(bufs[ug], sos[ug])

                start_word(jnp.minimum(c + 3, last), bufs[ug], sws[ug])
            return carry

        lax.fori_loop(0, n_chunks // 4, slot_body, 0)
        wait_word(bufs[1], sws[1])
        wait_word(bufs[2], sws[2])
        wait_pt(bufs[0], sps[0])
        wait_out(bufs[3], sos[3])

    return k


def kernel(input_ids, token_ids, W_word, W_type, W_pos, gamma, beta):
    B, L = input_ids.shape
    D = W_word.shape[1]
    N = B * L
    ids_flat = input_ids.reshape(N).astype(jnp.int32)
    tids_flat = token_ids.reshape(N).astype(jnp.int32)
    k = _build(N, D, L, n_workers=32)
    out = k(ids_flat, tids_flat, W_word, W_type, W_pos, gamma, beta)
    return out.reshape(B, L, D)


# speedup vs baseline: 1.0500x; 1.0500x over previous
import functools

import jax
import jax.numpy as jnp
from jax import lax
from jax.experimental import pallas as pl
from jax.experimental.pallas import tpu as pltpu
from jax.experimental.pallas import tpu_sc as plsc

_LANES = 16
_EPS = 1e-12
_UNROLL = 10

_DNUMS = lax.GatherDimensionNumbers(
    offset_dims=(), collapsed_slice_dims=(0,), start_index_map=(0,))


def _permute(v, perm):
    return lax.gather(v, perm.reshape(_LANES, 1), _DNUMS, (1,),
                      mode=lax.GatherScatterMode.PROMISE_IN_BOUNDS)


def _lane_sum(v):
    for k in (8, 4, 2, 1):
        v = v + _permute(v, lax.iota(jnp.int32, _LANES) ^ k)
    return v


def _rsqrt(v):
    i = lax.bitcast_convert_type(v, jnp.int32)
    i = jnp.int32(0x5F3759DF) - lax.shift_right_logical(i, 1)
    y = lax.bitcast_convert_type(i, jnp.float32)
    for _ in range(2):
        y = y * (1.5 - 0.5 * v * y * y)
    return y


def _build(N, D, L, n_workers):
    per_w = N // n_workers
    n_chunks = per_w // L
    n_pairs = n_chunks // 2
    n_sl = D // _LANES
    sub = [(st, min(128, L - st)) for st in range(0, L, 128)]
    mesh = plsc.VectorSubcoreMesh(core_axis_name="c", subcore_axis_name="s")
    info = plsc.get_sparse_core_info()
    NC, NS = info.num_cores, info.num_subcores

    @functools.partial(
        pl.kernel,
        mesh=mesh,
        out_type=jax.ShapeDtypeStruct((N, D), jnp.float32),
        scratch_types=[
            pltpu.VMEM((per_w,), jnp.int32),
            pltpu.VMEM((per_w,), jnp.int32),
            pltpu.VMEM((per_w,), jnp.int32),
            pltpu.VMEM((2, D), jnp.float32),
            pltpu.VMEM((L, D), jnp.float32),
            pltpu.VMEM((L, D), jnp.float32),
            pltpu.VMEM((L, D), jnp.float32),
            pltpu.VMEM((L, D), jnp.float32),
            pltpu.VMEM((D,), jnp.float32),
            pltpu.VMEM((D,), jnp.float32),
            pltpu.VMEM_SHARED((2 * L, D), jnp.float32),
            pltpu.SemaphoreType.DMA,
            pltpu.SemaphoreType.DMA,
            pltpu.SemaphoreType.DMA,
            pltpu.SemaphoreType.DMA,
            pltpu.SemaphoreType.DMA,
            pltpu.SemaphoreType.DMA,
            pltpu.SemaphoreType.DMA,
            pltpu.SemaphoreType.DMA,
            pltpu.SemaphoreType.DMA,
            pltpu.SemaphoreType.DMA,
            pltpu.SemaphoreType.DMA,
            pltpu.SemaphoreType.DMA,
        ],
    )
    def k(ids_hbm, tids_hbm, ww_hbm, wt_hbm, wp_hbm, g_hbm, b_hbm, out_hbm,
          widxv, tidxv, cidxv, wtypev, buf0, buf1, buf2, buf3, gv, bv, ptsh,
          sw0, sw1, sw2, sw3, sp0, sp1, sp2, sp3, so0, so1, so2, so3):
        bufs = (buf0, buf1, buf2, buf3)
        sws = (sw0, sw1, sw2, sw3)
        sps = (sp0, sp1, sp2, sp3)
        sos = (so0, so1, so2, so3)
        bufa = buf0
        sid = lax.axis_index("s")
        wid = sid * NC + lax.axis_index("c")
        base = wid * per_w
        pltpu.sync_copy(g_hbm, gv)
        pltpu.sync_copy(b_hbm, bv)
        pltpu.sync_copy(wt_hbm, wtypev)
        pltpu.sync_copy(ids_hbm.at[pl.ds(base, per_w)], widxv)
        pltpu.sync_copy(tids_hbm.at[pl.ds(base, per_w)], tidxv)

        n_blocks = (2 * L) // 8
        for kb in range((n_blocks + NS - 1) // NS):
            bb = sid + NS * kb

            @pl.when(bb < n_blocks)
            def _():
                t = (bb * 8) // L
                p0 = bb * 8 - t * L
                pltpu.sync_copy(wp_hbm.at[pl.ds(p0, 8)], bufa.at[pl.ds(0, 8)])
                for r in range(8):
                    for s in range(n_sl):
                        sl = pl.ds(s * _LANES, _LANES)
                        bufa[r, sl] = bufa[r, sl] + wtypev[t, sl]
                pltpu.sync_copy(bufa.at[pl.ds(0, 8)],
                                ptsh.at[pl.ds(bb * 8, 8)])

        def cidx_body(g, carry):
            g0 = g * _LANES
            pos = lax.rem(jnp.full((_LANES,), g0, jnp.int32)
                          + lax.iota(jnp.int32, _LANES), jnp.int32(L))
            cidxv[pl.ds(g0, _LANES)] = tidxv[pl.ds(g0, _LANES)] * L + pos
            return carry

        lax.fori_loop(0, per_w // _LANES, cidx_body, 0)
        plsc.subcore_barrier()

        def start_word(c, buf, sem):
            for st, ln in sub:
                pltpu.async_copy(
                    ww_hbm.at[widxv.at[pl.ds(c * L + st, ln)]],
                    buf.at[pl.ds(st, ln)], sem)

        def wait_word(buf, sem):
            for st, ln in sub:
                pltpu.make_async_copy(
                    ww_hbm.at[widxv.at[pl.ds(st, ln)]],
                    buf.at[pl.ds(st, ln)], sem).wait()

        def start_pt(c, buf, sem):
            for st, ln in sub:
                pltpu.async_copy(
                    ptsh.at[cidxv.at[pl.ds(c * L + st, ln)]],
                    buf.at[pl.ds(st, ln)], sem, add=True)

        def wait_pt(buf, sem):
            for st, ln in sub:
                pltpu.make_async_copy(
                    ptsh.at[cidxv.at[pl.ds(st, ln)]],
                    buf.at[pl.ds(st, ln)], sem).wait()

        def start_out(c, buf, sem):
            pltpu.async_copy(buf, out_hbm.at[pl.ds(base + c * L, L)], sem)

        def wait_out(buf, sem):
            pltpu.make_async_copy(buf, out_hbm.at[pl.ds(0, L)], sem).wait()

        def compute(buf):
            inv0 = tuple(
                ref[pl.ds(s * _LANES, _LANES)]
                for ref in (gv, bv) for s in range(n_sl))

            def tok_body(g, inv):
                gs, bs = inv[:n_sl], inv[n_sl:]
                for u in range(_UNROLL):
                    i = g * _UNROLL + u
                    sum_v = None
                    ssq_v = None
                    for s in range(n_sl):
                        sl = pl.ds(s * _LANES, _LANES)
                        x = buf[i, sl]
                        sum_v = x if s == 0 else sum_v + x
                        ssq_v = x * x if s == 0 else ssq_v + x * x
                    mean_v = _lane_sum(sum_v) * (1.0 / D)
                    var_v = _lane_sum(ssq_v) * (1.0 / D) - mean_v * mean_v
                    scale_v = _rsqrt(var_v + _EPS)
                    for s in range(n_sl):
                        sl = pl.ds(s * _LANES, _LANES)
                        buf[i, sl] = (buf[i, sl] - mean_v) * scale_v * gs[s] + bs[s]
                return inv

            lax.fori_loop(0, L // _UNROLL, tok_body, inv0)

        last = jnp.int32(n_chunks - 1)
        assert n_chunks % 4 == 0

        start_word(jnp.int32(0), bufs[0], sws[0])
        start_word(jnp.int32(1), bufs[1], sws[1])
        start_word(jnp.int32(2), bufs[2], sws[2])
        wait_word(bufs[0], sws[0])
        start_pt(jnp.int32(0), bufs[0], sps[0])

        def slot_body(t, carry):
            for u in range(4):
                c = t * 4 + u
                un, ug = (u + 1) % 4, (u + 3) % 4
                wait_word(bufs[un], sws[un])
                start_pt(jnp.minimum(c + 1, last), bufs[un], sps[un])
                wait_pt(bufs[u], sps[u])
                compute(bufs[u])
                start_out(c, bufs[u], sos[u])

                @pl.when(c >= 1)
                def _():
                    wait_out(bufs[ug], sos[ug])

                start_word(jnp.minimum(c + 3, last), bufs[ug], sws[ug])
            return carry

        lax.fori_loop(0, n_chunks // 4, slot_body, 0)
        wait_word(bufs[1], sws[1])
        wait_word(bufs[2], sws[2])
        wait_pt(bufs[0], sps[0])
        wait_out(bufs[3], sos[3])

    return k


def kernel(input_ids, token_ids, W_word, W_type, W_pos, gamma, beta):
    B, L = input_ids.shape
    D = W_word.shape[1]
    N = B * L
    ids_flat = input_ids.reshape(N).astype(jnp.int32)
    tids_flat = token_ids.reshape(N).astype(jnp.int32)
    k = _build(N, D, L, n_workers=32)
    out = k(ids_flat, tids_flat, W_word, W_type, W_pos, gamma, beta)
    return out.reshape(B, L, D)


# speedup vs baseline: 1.0874x; 1.0356x over previous
import functools

import jax
import jax.numpy as jnp
from jax import lax
from jax.experimental import pallas as pl
from jax.experimental.pallas import tpu as pltpu
from jax.experimental.pallas import tpu_sc as plsc

_LANES = 16
_EPS = 1e-12
_UNROLL = 8

_DNUMS = lax.GatherDimensionNumbers(
    offset_dims=(), collapsed_slice_dims=(0,), start_index_map=(0,))


def _permute(v, perm):
    return lax.gather(v, perm.reshape(_LANES, 1), _DNUMS, (1,),
                      mode=lax.GatherScatterMode.PROMISE_IN_BOUNDS)


def _lane_sum(v):
    for k in (8, 4, 2, 1):
        v = v + _permute(v, lax.iota(jnp.int32, _LANES) ^ k)
    return v


def _rsqrt(v):
    i = lax.bitcast_convert_type(v, jnp.int32)
    i = jnp.int32(0x5F3759DF) - lax.shift_right_logical(i, 1)
    y = lax.bitcast_convert_type(i, jnp.float32)
    for _ in range(2):
        y = y * (1.5 - 0.5 * v * y * y)
    return y


def _build(N, D, L, n_workers):
    per_w = N // n_workers
    n_chunks = per_w // L
    n_sl = D // _LANES
    sub = [(st, min(128, L - st)) for st in range(0, L, 128)]
    mesh = plsc.VectorSubcoreMesh(core_axis_name="c", subcore_axis_name="s")
    info = plsc.get_sparse_core_info()
    NC, NS = info.num_cores, info.num_subcores

    @functools.partial(
        pl.kernel,
        mesh=mesh,
        out_type=jax.ShapeDtypeStruct((N, D), jnp.float32),
        scratch_types=[
            pltpu.VMEM((per_w,), jnp.int32),
            pltpu.VMEM((per_w,), jnp.int32),
            pltpu.VMEM((per_w,), jnp.int32),
            pltpu.VMEM((2, D), jnp.float32),
            pltpu.VMEM((L, D), jnp.float32),
            pltpu.VMEM((L, D), jnp.float32),
            pltpu.VMEM((L, D), jnp.float32),
            pltpu.VMEM((L, D), jnp.float32),
            pltpu.VMEM((D,), jnp.float32),
            pltpu.VMEM((D,), jnp.float32),
            pltpu.VMEM_SHARED((2 * L, D), jnp.float32),
            pltpu.SemaphoreType.DMA,
            pltpu.SemaphoreType.DMA,
            pltpu.SemaphoreType.DMA,
            pltpu.SemaphoreType.DMA,
            pltpu.SemaphoreType.DMA,
            pltpu.SemaphoreType.DMA,
            pltpu.SemaphoreType.DMA,
            pltpu.SemaphoreType.DMA,
            pltpu.SemaphoreType.DMA,
            pltpu.SemaphoreType.DMA,
            pltpu.SemaphoreType.DMA,
            pltpu.SemaphoreType.DMA,
        ],
    )
    def k(ids_hbm, tids_hbm, ww_hbm, wt_hbm, wp_hbm, g_hbm, b_hbm, out_hbm,
          widxv, tidxv, cidxv, wtypev, buf0, buf1, buf2, buf3, gv, bv, ptsh,
          sw0, sw1, sw2, sw3, sp0, sp1, sp2, sp3, so0, so1, so2, so3):
        bufs = (buf0, buf1, buf2, buf3)
        sws = (sw0, sw1, sw2, sw3)
        sps = (sp0, sp1, sp2, sp3)
        sos = (so0, so1, so2, so3)
        bufa = buf0
        sid = lax.axis_index("s")
        wid = sid * NC + lax.axis_index("c")
        base = wid * per_w
        pltpu.sync_copy(g_hbm, gv)
        pltpu.sync_copy(b_hbm, bv)
        pltpu.sync_copy(wt_hbm, wtypev)
        pltpu.sync_copy(ids_hbm.at[pl.ds(base, per_w)], widxv)
        pltpu.sync_copy(tids_hbm.at[pl.ds(base, per_w)], tidxv)

        n_blocks = (2 * L) // 8
        for kb in range((n_blocks + NS - 1) // NS):
            bb = sid + NS * kb

            @pl.when(bb < n_blocks)
            def _():
                t = (bb * 8) // L
                p0 = bb * 8 - t * L
                pltpu.sync_copy(wp_hbm.at[pl.ds(p0, 8)], bufa.at[pl.ds(0, 8)])
                for r in range(8):
                    for s in range(n_sl):
                        sl = pl.ds(s * _LANES, _LANES)
                        bufa[r, sl] = bufa[r, sl] + wtypev[t, sl]
                pltpu.sync_copy(bufa.at[pl.ds(0, 8)],
                                ptsh.at[pl.ds(bb * 8, 8)])

        def cidx_body(g, carry):
            g0 = g * _LANES
            pos = lax.rem(jnp.full((_LANES,), g0, jnp.int32)
                          + lax.iota(jnp.int32, _LANES), jnp.int32(L))
            cidxv[pl.ds(g0, _LANES)] = tidxv[pl.ds(g0, _LANES)] * L + pos
            return carry

        lax.fori_loop(0, per_w // _LANES, cidx_body, 0)
        plsc.subcore_barrier()

        def start_word(c, buf, sem):
            for st, ln in sub:
                pltpu.async_copy(
                    ww_hbm.at[widxv.at[pl.ds(c * L + st, ln)]],
                    buf.at[pl.ds(st, ln)], sem)

        def wait_word(buf, sem):
            for st, ln in sub:
                pltpu.make_async_copy(
                    ww_hbm.at[widxv.at[pl.ds(st, ln)]],
                    buf.at[pl.ds(st, ln)], sem).wait()

        def start_pt(c, buf, sem):
            for st, ln in sub:
                pltpu.async_copy(
                    ptsh.at[cidxv.at[pl.ds(c * L + st, ln)]],
                    buf.at[pl.ds(st, ln)], sem, add=True)

        def wait_pt(buf, sem):
            for st, ln in sub:
                pltpu.make_async_copy(
                    ptsh.at[cidxv.at[pl.ds(st, ln)]],
                    buf.at[pl.ds(st, ln)], sem).wait()

        def start_out(c, buf, sem):
            pltpu.async_copy(buf, out_hbm.at[pl.ds(base + c * L, L)], sem)

        def wait_out(buf, sem):
            pltpu.make_async_copy(buf, out_hbm.at[pl.ds(0, L)], sem).wait()

        def compute(buf):
            inv0 = tuple(
                ref[pl.ds(s * _LANES, _LANES)]
                for ref in (gv, bv) for s in range(n_sl))

            def tok_body(g, inv):
                gs, bs = inv[:n_sl], inv[n_sl:]
                for u in range(_UNROLL):
                    i = g * _UNROLL + u
                    sum_v = None
                    ssq_v = None
                    for s in range(n_sl):
                        sl = pl.ds(s * _LANES, _LANES)
                        x = buf[i, sl]
                        sum_v = x if s == 0 else sum_v + x
                        ssq_v = x * x if s == 0 else ssq_v + x * x
                    mean_v = _lane_sum(sum_v) * (1.0 / D)
                    var_v = _lane_sum(ssq_v) * (1.0 / D) - mean_v * mean_v
                    scale_v = _rsqrt(var_v + _EPS)
                    for s in range(n_sl):
                        sl = pl.ds(s * _LANES, _LANES)
                        buf[i, sl] = (buf[i, sl] - mean_v) * scale_v * gs[s] + bs[s]
                return inv

            lax.fori_loop(0, L // _UNROLL, tok_body, inv0)

        last = jnp.int32(n_chunks - 1)
        assert n_chunks % 4 == 0

        start_word(jnp.int32(0), bufs[0], sws[0])
        start_word(jnp.int32(1), bufs[1], sws[1])
        start_word(jnp.int32(2), bufs[2], sws[2])
        wait_word(bufs[0], sws[0])
        start_pt(jnp.int32(0), bufs[0], sps[0])

        def slot_body(t, carry):
            for u in range(4):
                c = t * 4 + u
                un, ug = (u + 1) % 4, (u + 3) % 4
                wait_word(bufs[un], sws[un])
                start_pt(jnp.minimum(c + 1, last), bufs[un], sps[un])
                wait_pt(bufs[u], sps[u])
                compute(bufs[u])
                start_out(c, bufs[u], sos[u])

                @pl.when(c >= 1)
                def _():
                    wait_out(bufs[ug], sos[ug])

                start_word(jnp.minimum(c + 3, last), bufs[ug], sws[ug])
            return carry

        lax.fori_loop(0, n_chunks // 4, slot_body, 0)
        wait_word(bufs[1], sws[1])
        wait_word(bufs[2], sws[2])
        wait_pt(bufs[0], sps[0])
        wait_out(bufs[3], sos[3])

    return k


def kernel(input_ids, token_ids, W_word, W_type, W_pos, gamma, beta):
    B, L = input_ids.shape
    D = W_word.shape[1]
    N = B * L
    ids_flat = input_ids.reshape(N).astype(jnp.int32)
    tids_flat = token_ids.reshape(N).astype(jnp.int32)
    k = _build(N, D, L, n_workers=32)
    out = k(ids_flat, tids_flat, W_word, W_type, W_pos, gamma, beta)
    return out.reshape(B, L, D)


# speedup vs baseline: 1.0929x; 1.0051x over previous
import functools

import jax
import jax.numpy as jnp
from jax import lax
from jax.experimental import pallas as pl
from jax.experimental.pallas import tpu as pltpu
from jax.experimental.pallas import tpu_sc as plsc

_LANES = 16
_EPS = 1e-12
_UNROLL = 8

_DNUMS = lax.GatherDimensionNumbers(
    offset_dims=(), collapsed_slice_dims=(0,), start_index_map=(0,))


def _permute(v, perm):
    return lax.gather(v, perm.reshape(_LANES, 1), _DNUMS, (1,),
                      mode=lax.GatherScatterMode.PROMISE_IN_BOUNDS)


def _lane_sum(v):
    for k in (8, 4, 2, 1):
        v = v + _permute(v, lax.iota(jnp.int32, _LANES) ^ k)
    return v


def _rsqrt(v):
    i = lax.bitcast_convert_type(v, jnp.int32)
    i = jnp.int32(0x5F3759DF) - lax.shift_right_logical(i, 1)
    y = lax.bitcast_convert_type(i, jnp.float32)
    for _ in range(2):
        y = y * (1.5 - 0.5 * v * y * y)
    return y


def _build(N, D, L, n_workers):
    per_w = N // n_workers
    n_chunks = per_w // L
    n_sl = D // _LANES
    sub = [(st, min(128, L - st)) for st in range(0, L, 128)]
    mesh = plsc.VectorSubcoreMesh(core_axis_name="c", subcore_axis_name="s")
    info = plsc.get_sparse_core_info()
    NC, NS = info.num_cores, info.num_subcores

    @functools.partial(
        pl.kernel,
        mesh=mesh,
        out_type=jax.ShapeDtypeStruct((N, D), jnp.float32),
        scratch_types=[
            pltpu.VMEM((per_w,), jnp.int32),
            pltpu.VMEM((per_w,), jnp.int32),
            pltpu.VMEM((per_w,), jnp.int32),
            pltpu.VMEM((2, D), jnp.float32),
            pltpu.VMEM((L, D), jnp.float32),
            pltpu.VMEM((L, D), jnp.float32),
            pltpu.VMEM((L, D), jnp.float32),
            pltpu.VMEM((L, D), jnp.float32),
            pltpu.VMEM((D,), jnp.float32),
            pltpu.VMEM((D,), jnp.float32),
            pltpu.VMEM_SHARED((2 * L, D), jnp.float32),
            pltpu.SemaphoreType.DMA,
            pltpu.SemaphoreType.DMA,
            pltpu.SemaphoreType.DMA,
            pltpu.SemaphoreType.DMA,
            pltpu.SemaphoreType.DMA,
            pltpu.SemaphoreType.DMA,
            pltpu.SemaphoreType.DMA,
            pltpu.SemaphoreType.DMA,
            pltpu.SemaphoreType.DMA,
            pltpu.SemaphoreType.DMA,
            pltpu.SemaphoreType.DMA,
            pltpu.SemaphoreType.DMA,
        ],
    )
    def k(ids_hbm, tids_hbm, ww_hbm, wt_hbm, wp_hbm, g_hbm, b_hbm, out_hbm,
          widxv, tidxv, cidxv, wtypev, buf0, buf1, buf2, buf3, gv, bv, ptsh,
          sw0, sw1, sw2, sw3, sp0, sp1, sp2, sp3, so0, so1, so2, so3):
        bufs = (buf0, buf1, buf2, buf3)
        sws = (sw0, sw1, sw2, sw3)
        sps = (sp0, sp1, sp2, sp3)
        sos = (so0, so1, so2, so3)
        bufa = buf0
        sid = lax.axis_index("s")
        wid = sid * NC + lax.axis_index("c")
        base = wid * per_w
        pltpu.sync_copy(g_hbm, gv)
        pltpu.sync_copy(b_hbm, bv)
        pltpu.sync_copy(wt_hbm, wtypev)
        pltpu.sync_copy(ids_hbm.at[pl.ds(base, per_w)], widxv)
        pltpu.sync_copy(tids_hbm.at[pl.ds(base, per_w)], tidxv)

        n_blocks = (2 * L) // 8
        for kb in range((n_blocks + NS - 1) // NS):
            bb = sid + NS * kb

            @pl.when(bb < n_blocks)
            def _():
                t = (bb * 8) // L
                p0 = bb * 8 - t * L
                pltpu.sync_copy(wp_hbm.at[pl.ds(p0, 8)], bufa.at[pl.ds(0, 8)])
                for r in range(8):
                    for s in range(n_sl):
                        sl = pl.ds(s * _LANES, _LANES)
                        bufa[r, sl] = bufa[r, sl] + wtypev[t, sl]
                pltpu.sync_copy(bufa.at[pl.ds(0, 8)],
                                ptsh.at[pl.ds(bb * 8, 8)])

        def cidx_body(g, carry):
            g0 = g * _LANES
            pos = lax.rem(jnp.full((_LANES,), g0, jnp.int32)
                          + lax.iota(jnp.int32, _LANES), jnp.int32(L))
            cidxv[pl.ds(g0, _LANES)] = tidxv[pl.ds(g0, _LANES)] * L + pos
            return carry

        lax.fori_loop(0, per_w // _LANES, cidx_body, 0)
        plsc.subcore_barrier()

        def start_word(c, buf, sem):
            for st, ln in sub:
                pltpu.async_copy(
                    ww_hbm.at[widxv.at[pl.ds(c * L + st, ln)]],
                    buf.at[pl.ds(st, ln)], sem)

        def wait_word(buf, sem):
            for st, ln in sub:
                pltpu.make_async_copy(
                    ww_hbm.at[widxv.at[pl.ds(st, ln)]],
                    buf.at[pl.ds(st, ln)], sem).wait()

        def start_pt(c, buf, sem):
            for st, ln in sub:
                pltpu.async_copy(
                    ptsh.at[cidxv.at[pl.ds(c * L + st, ln)]],
                    buf.at[pl.ds(st, ln)], sem, add=True)

        def wait_pt(buf, sem):
            for st, ln in sub:
                pltpu.make_async_copy(
                    ptsh.at[cidxv.at[pl.ds(st, ln)]],
                    buf.at[pl.ds(st, ln)], sem).wait()

        def start_out(c, buf, sem):
            pltpu.async_copy(buf, out_hbm.at[pl.ds(base + c * L, L)], sem)

        def wait_out(buf, sem):
            pltpu.make_async_copy(buf, out_hbm.at[pl.ds(0, L)], sem).wait()

        def compute(buf):
            inv0 = tuple(
                ref[pl.ds(s * _LANES, _LANES)]
                for ref in (gv, bv) for s in range(n_sl))

            def tok_body(g, inv):
                gs, bs = inv[:n_sl], inv[n_sl:]
                for u in range(_UNROLL):
                    i = g * _UNROLL + u
                    sum_v = None
                    ssq_v = None
                    for s in range(n_sl):
                        sl = pl.ds(s * _LANES, _LANES)
                        x = buf[i, sl]
                        sum_v = x if s == 0 else sum_v + x
                        ssq_v = x * x if s == 0 else ssq_v + x * x
                    mean_v = _lane_sum(sum_v) * (1.0 / D)
                    var_v = _lane_sum(ssq_v) * (1.0 / D) - mean_v * mean_v
                    scale_v = _rsqrt(var_v + _EPS)
                    for s in range(n_sl):
                        sl = pl.ds(s * _LANES, _LANES)
                        buf[i, sl] = (buf[i, sl] - mean_v) * scale_v * gs[s] + bs[s]
                return inv

            lax.fori_loop(0, L // _UNROLL, tok_body, inv0)

        last = jnp.int32(n_chunks - 1)
        assert n_chunks % 4 == 0

        start_word(jnp.int32(0), bufs[0], sws[0])
        start_word(jnp.int32(1), bufs[1], sws[1])
        start_word(jnp.int32(2), bufs[2], sws[2])
        wait_word(bufs[0], sws[0])
        start_pt(jnp.int32(0), bufs[0], sps[0])

        def slot_body(t, carry):
            for u in range(4):
                c = t * 4 + u
                un, ug = (u + 1) % 4, (u + 3) % 4
                wait_pt(bufs[u], sps[u])
                wait_word(bufs[un], sws[un])
                start_pt(jnp.minimum(c + 1, last), bufs[un], sps[un])
                compute(bufs[u])
                start_out(c, bufs[u], sos[u])

                @pl.when(c >= 1)
                def _():
                    wait_out(bufs[ug], sos[ug])

                start_word(jnp.minimum(c + 3, last), bufs[ug], sws[ug])
            return carry

        lax.fori_loop(0, n_chunks // 4, slot_body, 0)
        wait_word(bufs[1], sws[1])
        wait_word(bufs[2], sws[2])
        wait_pt(bufs[0], sps[0])
        wait_out(bufs[3], sos[3])

    return k


def kernel(input_ids, token_ids, W_word, W_type, W_pos, gamma, beta):
    B, L = input_ids.shape
    D = W_word.shape[1]
    N = B * L
    ids_flat = input_ids.reshape(N).astype(jnp.int32)
    tids_flat = token_ids.reshape(N).astype(jnp.int32)
    k = _build(N, D, L, n_workers=32)
    out = k(ids_flat, tids_flat, W_word, W_type, W_pos, gamma, beta)
    return out.reshape(B, L, D)
